# Initial kernel scaffold; baseline (speedup 1.0000x reference)
import jax
import jax.numpy as jnp
from jax.experimental import pallas as pl


def kernel(points_coords, centers_coords, temb, points_features):
    raise NotImplementedError("write your pallas kernel here")


# speedup vs baseline: 69.7579x; 69.7579x over previous
import dataclasses
import functools

import jax
import jax.numpy as jnp
from jax import lax
from jax.experimental import pallas as pl
from jax.experimental.pallas import tpu as pltpu
from jax.experimental.pallas import tpu_sc as plsc

B, M, N, K = 4, 1024, 4096, 32
CF = 64
GPB = 8
MPG = M // GPB
CPG = CF // GPB


def _ball_query_jnp(centers_coords, points_coords):
    centers = jnp.transpose(centers_coords, (0, 2, 1))
    points = jnp.transpose(points_coords, (0, 2, 1))
    c2 = jnp.sum(centers ** 2, axis=-1)
    p2 = jnp.sum(points ** 2, axis=-1)
    dist_sq = (c2[:, :, None] + p2[:, None, :]
               - 2.0 * jnp.einsum('bmd,bnd->bmn', centers, points))
    _, idx = lax.top_k(-dist_sq, K)
    return idx.astype(jnp.int32)


def _group_body(pc_hbm, cc_hbm, temb_hbm, pf_hbm, idx_hbm,
                out1_hbm, out2_hbm,
                idxv, pcv, ccv, row_v, featout, co):
    c = lax.axis_index("c")
    s = lax.axis_index("s")
    wid = s * 2 + c
    b = wid // GPB
    g = wid % GPB

    pltpu.sync_copy(idx_hbm.at[b], idxv)

    def do_tensor(src_hbm, out_hbm, ch_off):
        @pl.loop(0, CPG)
        def _ch(ci):
            ch = g * CPG + ci
            pltpu.sync_copy(src_hbm.at[b, ch], row_v)

            @pl.loop(0, M)
            def _row(r):
                i0 = idxv[pl.ds(r * K, 16)]
                i1 = idxv[pl.ds(r * K + 16, 16)]
                featout[pl.ds(r * K, 16)] = plsc.load_gather(row_v, [i0])
                featout[pl.ds(r * K + 16, 16)] = plsc.load_gather(row_v, [i1])

            pltpu.sync_copy(featout, out_hbm.at[b, ch_off + ch])

    do_tensor(pf_hbm, out1_hbm, 3)
    do_tensor(temb_hbm, out2_hbm, 0)

    pltpu.sync_copy(pc_hbm.at[b], pcv)
    for d in range(3):
        pltpu.sync_copy(cc_hbm.at[b, pl.ds(d * M + g * MPG, MPG)],
                        ccv.at[pl.ds(d * MPG, MPG)])

    for d in range(3):
        @pl.loop(0, MPG)
        def _crow(r):
            m = g * MPG + r
            i0 = idxv[pl.ds(m * K, 16)] + (d * N)
            i1 = idxv[pl.ds(m * K + 16, 16)] + (d * N)
            r_vec = jnp.zeros((16,), jnp.int32) + (r + d * MPG)
            cvec = plsc.load_gather(ccv, [r_vec])
            co[pl.ds(r * K, 16)] = plsc.load_gather(pcv, [i0]) - cvec
            co[pl.ds(r * K + 16, 16)] = plsc.load_gather(pcv, [i1]) - cvec

        pltpu.sync_copy(co, out1_hbm.at[b, d, pl.ds(g * MPG * K, MPG * K)])


@jax.jit
def _grouping_sc(points_coords, centers_coords, temb, points_features, idx):
    mesh = plsc.VectorSubcoreMesh(core_axis_name="c", subcore_axis_name="s")
    f32 = jnp.float32
    cp = pltpu.CompilerParams()
    if "needs_layout_passes" in pltpu.CompilerParams.__dataclass_fields__:
        cp = dataclasses.replace(cp, needs_layout_passes=False)
    run = pl.kernel(
        _group_body,
        compiler_params=cp,
        out_type=(jax.ShapeDtypeStruct((B, 3 + CF, M * K), f32),
                  jax.ShapeDtypeStruct((B, CF, M * K), f32)),
        mesh=mesh,
        scratch_types=[
            pltpu.VMEM((M * K,), jnp.int32),
            pltpu.VMEM((3 * N,), f32),
            pltpu.VMEM((3 * MPG,), f32),
            pltpu.VMEM((N,), f32),
            pltpu.VMEM((M * K,), f32),
            pltpu.VMEM((MPG * K,), f32),
        ],
    )
    out1, out2 = run(points_coords.reshape(B, 3 * N),
                     centers_coords.reshape(B, 3 * M), temb,
                     points_features, idx)
    return out1.reshape(B, 3 + CF, M, K), out2.reshape(B, CF, M, K)


def kernel(points_coords, centers_coords, temb, points_features):
    idx = _ball_query_jnp(centers_coords, points_coords)
    idx_flat = idx.reshape(B, M * K)
    return _grouping_sc(points_coords, centers_coords, temb,
                        points_features, idx_flat)


# speedup vs baseline: 155.1805x; 2.2246x over previous
import dataclasses
import functools

import jax
import jax.numpy as jnp
from jax import lax
from jax.experimental import pallas as pl
from jax.experimental.pallas import tpu as pltpu
from jax.experimental.pallas import tpu_sc as plsc

B, M, N, K = 4, 1024, 4096, 32
CF = 64
GPB = 8
MPG = M // GPB
CPG = CF // GPB


_NCHUNK = N // 16
_INF = float("inf")


def _merge_top32(d0, i0, d1, i1, cd, ci):
    rb = lax.rev(cd, (0,))
    rbi = lax.rev(ci, (0,))
    m = d1 <= rb
    l1 = jnp.where(m, d1, rb)
    l1i = jnp.where(m, i1, rbi)
    m2 = d0 <= l1
    p = jnp.where(m2, d0, l1)
    pi = jnp.where(m2, i0, l1i)
    q = jnp.where(m2, l1, d0)
    qi = jnp.where(m2, l1i, i0)
    p_s, pi_s = plsc.sort_key_val(p, pi)
    q_s, qi_s = plsc.sort_key_val(q, qi)
    return p_s, pi_s, q_s, qi_s


def _round_bf16(x):
    bits = plsc.bitcast(x, jnp.int32)
    lsb = lax.shift_right_logical(bits, 16) & 1
    rounded = (bits + 0x7FFF + lsb) & jnp.int32(-65536)
    return plsc.bitcast(rounded, jnp.float32)


def _two_sum(a, b):
    s = a + b
    bp = s - a
    err = (a - (s - bp)) + (b - bp)
    return s, err


def _sum3_single_round(t0, t1, t2):
    s1, e1 = _two_sum(t1, t2)
    s2, e2 = _two_sum(t0, s1)
    return s2 + (e1 + e2)


def _tie_repair(d0, i0, d1, i1, kbuf, ibuf, iota16):
    kbuf[pl.ds(0, 16)] = d0
    kbuf[pl.ds(16, 16)] = d1
    ibuf[pl.ds(0, 16)] = i0
    ibuf[pl.ds(16, 16)] = i1
    prev_ix = jnp.maximum(iota16 - 1, 0)
    next_ix0 = iota16 + 1
    kp0 = plsc.load_gather(kbuf, [prev_ix])
    kn0 = plsc.load_gather(kbuf, [next_ix0])
    ip0 = plsc.load_gather(ibuf, [prev_ix])
    in0 = plsc.load_gather(ibuf, [next_ix0])
    eqp0 = (d0 == kp0) & (iota16 > 0)
    eqn0 = d0 == kn0
    ni0 = jnp.where(eqp0, jnp.maximum(i0, ip0),
                    jnp.where(eqn0, jnp.minimum(i0, in0), i0))
    prev_ix1 = iota16 + 15
    next_ix1 = jnp.minimum(iota16 + 17, 31)
    kp1 = plsc.load_gather(kbuf, [prev_ix1])
    kn1 = plsc.load_gather(kbuf, [next_ix1])
    ip1 = plsc.load_gather(ibuf, [prev_ix1])
    in1 = plsc.load_gather(ibuf, [next_ix1])
    eqp1 = d1 == kp1
    eqn1 = (d1 == kn1) & (iota16 < 15)
    ni1 = jnp.where(eqp1, jnp.maximum(i1, ip1),
                    jnp.where(eqn1, jnp.minimum(i1, in1), i1))
    return ni0, ni1


def _bq_body(pc_hbm, cc_hbm, idx_out_hbm, pcv, pbv, ccv, p2v, idxout, kbuf, ibuf):
    c = lax.axis_index("c")
    s = lax.axis_index("s")
    wid = s * 2 + c
    b = wid // GPB
    g = wid % GPB

    pltpu.sync_copy(pc_hbm.at[b], pcv)
    for d in range(3):
        pltpu.sync_copy(cc_hbm.at[b, pl.ds(d * M + g * MPG, MPG)],
                        ccv.at[pl.ds(d * MPG, MPG)])

    @pl.loop(0, _NCHUNK)
    def _p2(cix):
        px = pcv[pl.ds(cix * 16, 16)]
        py = pcv[pl.ds(N + cix * 16, 16)]
        pz = pcv[pl.ds(2 * N + cix * 16, 16)]
        p2v[pl.ds(cix * 16, 16)] = (px * px + py * py) + pz * pz
        pbv[pl.ds(cix * 16, 16)] = _round_bf16(px)
        pbv[pl.ds(N + cix * 16, 16)] = _round_bf16(py)
        pbv[pl.ds(2 * N + cix * 16, 16)] = _round_bf16(pz)

    zeros_i = jnp.zeros((16,), jnp.int32)
    iota16 = lax.iota(jnp.int32, 16)

    @pl.loop(0, MPG)
    def _row(r):
        r_splat = zeros_i + r
        cx = plsc.load_gather(ccv, [r_splat])
        cy = plsc.load_gather(ccv, [r_splat + MPG])
        cz = plsc.load_gather(ccv, [r_splat + 2 * MPG])
        c2 = (cx * cx + cy * cy) + cz * cz
        cxb = _round_bf16(cx)
        cyb = _round_bf16(cy)
        czb = _round_bf16(cz)

        init = (jnp.full((16,), _INF, jnp.float32), zeros_i,
                jnp.full((16,), _INF, jnp.float32), zeros_i,
                jnp.full((16,), _INF, jnp.float32))

        def chunk_step(cix, carry):
            d0, i0, d1, i1, tvec = carry
            px = pbv[pl.ds(cix * 16, 16)]
            py = pbv[pl.ds(N + cix * 16, 16)]
            pz = pbv[pl.ds(2 * N + cix * 16, 16)]
            p2 = p2v[pl.ds(cix * 16, 16)]
            e = _sum3_single_round(cxb * px, cyb * py, czb * pz)
            dd = (c2 + p2) - 2.0 * e
            mask = dd < tvec

            def with_merge(args):
                d0, i0, d1, i1, _ = args
                cd_raw = jnp.where(mask, dd, jnp.float32(_INF))
                cd, ci = plsc.sort_key_val(cd_raw, iota16 + cix * 16)
                nd0, ni0, nd1, ni1 = _merge_top32(d0, i0, d1, i1, cd, ci)
                nt = jnp.zeros((16,), jnp.float32) + jnp.max(nd1)
                return nd0, ni0, nd1, ni1, nt

            return lax.cond(jnp.any(mask), with_merge, lambda a: a, carry)

        d0, i0, d1, i1, _ = lax.fori_loop(0, _NCHUNK, chunk_step, init)
        i0, i1 = _tie_repair(d0, i0, d1, i1, kbuf, ibuf, iota16)
        idxout[pl.ds(r * K, 16)] = i0
        idxout[pl.ds(r * K + 16, 16)] = i1

    pltpu.sync_copy(idxout, idx_out_hbm.at[b, pl.ds(g * MPG * K, MPG * K)])


@jax.jit
def _ball_query_sc(points_coords_flat, centers_coords_flat):
    mesh = plsc.VectorSubcoreMesh(core_axis_name="c", subcore_axis_name="s")
    f32 = jnp.float32
    cp = pltpu.CompilerParams()
    if "needs_layout_passes" in pltpu.CompilerParams.__dataclass_fields__:
        cp = dataclasses.replace(cp, needs_layout_passes=False)
    run = pl.kernel(
        _bq_body,
        compiler_params=cp,
        out_type=jax.ShapeDtypeStruct((B, M * K), jnp.int32),
        mesh=mesh,
        scratch_types=[
            pltpu.VMEM((3 * N,), f32),
            pltpu.VMEM((3 * N,), f32),
            pltpu.VMEM((3 * MPG,), f32),
            pltpu.VMEM((N,), f32),
            pltpu.VMEM((MPG * K,), jnp.int32),
            pltpu.VMEM((32,), f32),
            pltpu.VMEM((32,), jnp.int32),
        ],
    )
    return run(points_coords_flat, centers_coords_flat)


def _group_body(pc_hbm, cc_hbm, temb_hbm, pf_hbm, idx_hbm,
                out1_hbm, out2_hbm,
                idxv, pcv, ccv, row_v, featout, co):
    c = lax.axis_index("c")
    s = lax.axis_index("s")
    wid = s * 2 + c
    b = wid // GPB
    g = wid % GPB

    pltpu.sync_copy(idx_hbm.at[b], idxv)

    def do_tensor(src_hbm, out_hbm, ch_off):
        @pl.loop(0, CPG)
        def _ch(ci):
            ch = g * CPG + ci
            pltpu.sync_copy(src_hbm.at[b, ch], row_v)

            @pl.loop(0, M)
            def _row(r):
                i0 = idxv[pl.ds(r * K, 16)]
                i1 = idxv[pl.ds(r * K + 16, 16)]
                featout[pl.ds(r * K, 16)] = plsc.load_gather(row_v, [i0])
                featout[pl.ds(r * K + 16, 16)] = plsc.load_gather(row_v, [i1])

            pltpu.sync_copy(featout, out_hbm.at[b, ch_off + ch])

    do_tensor(pf_hbm, out1_hbm, 3)
    do_tensor(temb_hbm, out2_hbm, 0)

    pltpu.sync_copy(pc_hbm.at[b], pcv)
    for d in range(3):
        pltpu.sync_copy(cc_hbm.at[b, pl.ds(d * M + g * MPG, MPG)],
                        ccv.at[pl.ds(d * MPG, MPG)])

    for d in range(3):
        @pl.loop(0, MPG)
        def _crow(r):
            m = g * MPG + r
            i0 = idxv[pl.ds(m * K, 16)] + (d * N)
            i1 = idxv[pl.ds(m * K + 16, 16)] + (d * N)
            r_vec = jnp.zeros((16,), jnp.int32) + (r + d * MPG)
            cvec = plsc.load_gather(ccv, [r_vec])
            co[pl.ds(r * K, 16)] = plsc.load_gather(pcv, [i0]) - cvec
            co[pl.ds(r * K + 16, 16)] = plsc.load_gather(pcv, [i1]) - cvec

        pltpu.sync_copy(co, out1_hbm.at[b, d, pl.ds(g * MPG * K, MPG * K)])


@jax.jit
def _grouping_sc(points_coords, centers_coords, temb, points_features, idx):
    mesh = plsc.VectorSubcoreMesh(core_axis_name="c", subcore_axis_name="s")
    f32 = jnp.float32
    cp = pltpu.CompilerParams()
    if "needs_layout_passes" in pltpu.CompilerParams.__dataclass_fields__:
        cp = dataclasses.replace(cp, needs_layout_passes=False)
    run = pl.kernel(
        _group_body,
        compiler_params=cp,
        out_type=(jax.ShapeDtypeStruct((B, 3 + CF, M * K), f32),
                  jax.ShapeDtypeStruct((B, CF, M * K), f32)),
        mesh=mesh,
        scratch_types=[
            pltpu.VMEM((M * K,), jnp.int32),
            pltpu.VMEM((3 * N,), f32),
            pltpu.VMEM((3 * MPG,), f32),
            pltpu.VMEM((N,), f32),
            pltpu.VMEM((M * K,), f32),
            pltpu.VMEM((MPG * K,), f32),
        ],
    )
    out1, out2 = run(points_coords.reshape(B, 3 * N),
                     centers_coords.reshape(B, 3 * M), temb,
                     points_features, idx)
    return out1.reshape(B, 3 + CF, M, K), out2.reshape(B, CF, M, K)


def kernel(points_coords, centers_coords, temb, points_features):
    pc_flat = points_coords.reshape(B, 3 * N)
    cc_flat = centers_coords.reshape(B, 3 * M)
    idx_flat = _ball_query_sc(pc_flat, cc_flat)
    return _grouping_sc(points_coords, centers_coords, temb,
                        points_features, idx_flat)
